# Initial kernel scaffold; baseline (speedup 1.0000x reference)
import jax
import jax.numpy as jnp
from jax.experimental import pallas as pl


def kernel(x, edge_index, edge_label_index, W1, b1, W2, b2, W3, b3, Wl1, bl1, Wl2, bl2):
    raise NotImplementedError("write your pallas kernel here")


# speedup vs baseline: 10.6536x; 10.6536x over previous
import functools

import jax
import jax.numpy as jnp
from jax import lax
from jax.experimental import pallas as pl
from jax.experimental.pallas import tpu as pltpu
from jax.experimental.pallas import tpu_sc as plsc

N = 10000
NP = 10240
E = 320000
PE = 327680
EL = 320000
PEL = 327680
F_IN = 128
H = 64
OUT = 32

NC = 2
NS = 16
NW = NC * NS
CH = 128
NCHUNK = PE // (NW * CH)
RPT = NP // NS

_MESH = plsc.VectorSubcoreMesh(core_axis_name="c", subcore_axis_name="s",
                               num_cores=NC, num_subcores=NS)
_SC_PARAMS = pltpu.CompilerParams(use_tc_tiling_on_sc=False)


def _fill_zeros(ref, nrows, ncols):
  z16 = jnp.zeros((16,), jnp.float32)
  def body(i, _):
    for j in range(ncols // 16):
      ref[i, pl.ds(j * 16, 16)] = z16
    return 0
  lax.fori_loop(0, nrows, body, 0)


@functools.partial(
    pl.kernel,
    out_type=jax.ShapeDtypeStruct((NC, NP), jnp.float32),
    mesh=_MESH,
    compiler_params=_SC_PARAMS,
    scratch_types=[
        pltpu.VMEM((NCHUNK, CH), jnp.int32),
        pltpu.VMEM((CH,), jnp.float32),
        pltpu.VMEM((RPT,), jnp.float32),
        pltpu.VMEM_SHARED((NP,), jnp.float32),
        pltpu.SemaphoreType.DMA,
    ],
)
def _deg_kernel(dst_hbm, out_hbm, dst_v, ones_v, zb_v, acc, sem):
  c = lax.axis_index("c")
  s = lax.axis_index("s")
  wid = s * NC + c
  pltpu.sync_copy(dst_hbm.at[wid], dst_v)
  one16 = jnp.ones((16,), jnp.float32)
  z16 = jnp.zeros((16,), jnp.float32)
  for j in range(CH // 16):
    ones_v[pl.ds(j * 16, 16)] = one16
  def zbody(i, _):
    zb_v[pl.ds(i * 16, 16)] = z16
    return 0
  lax.fori_loop(0, RPT // 16, zbody, 0)
  pltpu.sync_copy(zb_v, acc.at[pl.ds(s * RPT, RPT)])
  plsc.subcore_barrier()
  def body(j, _):
    pltpu.sync_copy(ones_v, acc.at[dst_v.at[j]], add=True)
    return 0
  lax.fori_loop(0, NCHUNK, body, 0)
  plsc.subcore_barrier()
  pltpu.sync_copy(acc.at[pl.ds(s * RPT, RPT)], zb_v)
  pltpu.sync_copy(zb_v, out_hbm.at[c, pl.ds(s * RPT, RPT)])


def _make_spmm(F):
  @functools.partial(
      pl.kernel,
      out_type=jax.ShapeDtypeStruct((NC, NP, F), jnp.float32),
      mesh=_MESH,
      compiler_params=_SC_PARAMS,
      scratch_types=[
          pltpu.VMEM((NCHUNK, CH), jnp.int32),
          pltpu.VMEM((NCHUNK, CH), jnp.int32),
          pltpu.VMEM((CH, F), jnp.float32),
          pltpu.VMEM((RPT, F), jnp.float32),
          pltpu.VMEM_SHARED((NP, F), jnp.float32),
          pltpu.SemaphoreType.DMA,
      ],
  )
  def spmm(h_hbm, src_hbm, dst_hbm, out_hbm, src_v, dst_v, rows_v, zb_v, acc,
           sem):
    c = lax.axis_index("c")
    s = lax.axis_index("s")
    wid = s * NC + c
    pltpu.sync_copy(src_hbm.at[wid], src_v)
    pltpu.sync_copy(dst_hbm.at[wid], dst_v)
    _fill_zeros(zb_v, CH, F)
    for k in range(RPT // CH):
      pltpu.sync_copy(zb_v.at[pl.ds(0, CH)],
                      acc.at[pl.ds(s * RPT + k * CH, CH)])
    plsc.subcore_barrier()
    def body(j, _):
      pltpu.async_copy(h_hbm.at[src_v.at[j]], rows_v, sem).wait()
      pltpu.sync_copy(rows_v, acc.at[dst_v.at[j]], add=True)
      return 0
    lax.fori_loop(0, NCHUNK, body, 0)
    plsc.subcore_barrier()
    pltpu.sync_copy(acc.at[pl.ds(s * RPT, RPT)], zb_v)
    pltpu.sync_copy(zb_v, out_hbm.at[c, pl.ds(s * RPT, RPT)])
  return spmm


_spmm64 = _make_spmm(H)
_spmm32 = _make_spmm(OUT)


@functools.partial(
    pl.kernel,
    out_type=(jax.ShapeDtypeStruct((PEL, OUT), jnp.float32),
              jax.ShapeDtypeStruct((PEL, OUT), jnp.float32)),
    mesh=_MESH,
    compiler_params=_SC_PARAMS,
    scratch_types=[
        pltpu.VMEM((NCHUNK, CH), jnp.int32),
        pltpu.VMEM((NCHUNK, CH), jnp.int32),
        pltpu.VMEM((CH, OUT), jnp.float32),
        pltpu.VMEM((CH, OUT), jnp.float32),
        pltpu.SemaphoreType.DMA,
    ],
)
def _decode_gather(z_hbm, row_hbm, col_hbm, efl_hbm, efr_hbm,
                   row_v, col_v, bufl, bufr, sem):
  c = lax.axis_index("c")
  s = lax.axis_index("s")
  wid = s * NC + c
  base = wid * (NCHUNK * CH)
  pltpu.sync_copy(row_hbm.at[wid], row_v)
  pltpu.sync_copy(col_hbm.at[wid], col_v)
  def body(j, _):
    cp1 = pltpu.async_copy(z_hbm.at[row_v.at[j]], bufl, sem)
    cp2 = pltpu.async_copy(z_hbm.at[col_v.at[j]], bufr, sem)
    cp1.wait()
    cp2.wait()
    pltpu.sync_copy(bufl, efl_hbm.at[pl.ds(base + j * CH, CH)])
    pltpu.sync_copy(bufr, efr_hbm.at[pl.ds(base + j * CH, CH)])
    return 0
  lax.fori_loop(0, NCHUNK, body, 0)


_BM = 1024
_GRID = NP // _BM


def _dinv_of(degp_ref):
  deg = degp_ref[0] + degp_ref[1] + 1.0
  return lax.rsqrt(deg)


def _b1_body(x_ref, w1_ref, degp_ref, h_ref):
  dinv = _dinv_of(degp_ref)
  h = jnp.dot(x_ref[...], w1_ref[...], preferred_element_type=jnp.float32)
  h_ref[...] = h * dinv


def _mid_body(p_ref, hprev_ref, degp_ref, b_ref, w_ref, hnext_ref):
  dinv = _dinv_of(degp_ref)
  srt = p_ref[0] + p_ref[1] + hprev_ref[...]
  z = jnp.maximum(srt * dinv + b_ref[0:1, :], 0.0)
  hnext_ref[...] = jnp.dot(z, w_ref[...],
                           preferred_element_type=jnp.float32) * dinv


def _b4_body(p_ref, hprev_ref, degp_ref, b_ref, z_ref):
  dinv = _dinv_of(degp_ref)
  z_ref[...] = (p_ref[0] + p_ref[1] + hprev_ref[...]) * dinv + b_ref[0:1, :]


def _bspec(shape, im):
  return pl.BlockSpec(shape, im)


def _tc_b1(x_pad, W1, degp3):
  return pl.pallas_call(
      _b1_body,
      grid=(_GRID,),
      in_specs=[
          _bspec((_BM, F_IN), lambda i: (i, 0)),
          _bspec((F_IN, H), lambda i: (0, 0)),
          _bspec((NC, _BM, 1), lambda i: (0, i, 0)),
      ],
      out_specs=_bspec((_BM, H), lambda i: (i, 0)),
      out_shape=jax.ShapeDtypeStruct((NP, H), jnp.float32),
  )(x_pad, W1, degp3)


def _tc_mid(P, hprev, degp3, b2d, W, fout):
  fin = hprev.shape[1]
  return pl.pallas_call(
      _mid_body,
      grid=(_GRID,),
      in_specs=[
          _bspec((NC, _BM, fin), lambda i: (0, i, 0)),
          _bspec((_BM, fin), lambda i: (i, 0)),
          _bspec((NC, _BM, 1), lambda i: (0, i, 0)),
          _bspec((8, fin), lambda i: (0, 0)),
          _bspec((fin, fout), lambda i: (0, 0)),
      ],
      out_specs=_bspec((_BM, fout), lambda i: (i, 0)),
      out_shape=jax.ShapeDtypeStruct((NP, fout), jnp.float32),
  )(P, hprev, degp3, b2d, W)


def _tc_b4(P, hprev, degp3, b2d):
  return pl.pallas_call(
      _b4_body,
      grid=(_GRID,),
      in_specs=[
          _bspec((NC, _BM, OUT), lambda i: (0, i, 0)),
          _bspec((_BM, OUT), lambda i: (i, 0)),
          _bspec((NC, _BM, 1), lambda i: (0, i, 0)),
          _bspec((8, OUT), lambda i: (0, 0)),
      ],
      out_specs=_bspec((_BM, OUT), lambda i: (i, 0)),
      out_shape=jax.ShapeDtypeStruct((NP, OUT), jnp.float32),
  )(P, hprev, degp3, b2d)


_EBM = 2048
_EGRID = PEL // _EBM


def _mlp_body(efl_ref, efr_ref, a1_ref, a2_ref, b1_ref, w2_ref, b2_ref,
              out_ref):
  hl = jnp.dot(efl_ref[...], a1_ref[...], preferred_element_type=jnp.float32)
  hr = jnp.dot(efr_ref[...], a2_ref[...], preferred_element_type=jnp.float32)
  hh = jnp.maximum(hl + hr + b1_ref[0:1, :], 0.0)
  w = w2_ref[0:1, :]
  out_ref[...] = jnp.sum(hh * w, axis=1, keepdims=True) + b2_ref[0, 0]


def _tc_mlp(efl, efr, A1, A2, bl1t, wl2t, bl2t):
  return pl.pallas_call(
      _mlp_body,
      grid=(_EGRID,),
      in_specs=[
          _bspec((_EBM, OUT), lambda i: (i, 0)),
          _bspec((_EBM, OUT), lambda i: (i, 0)),
          _bspec((OUT, H), lambda i: (0, 0)),
          _bspec((OUT, H), lambda i: (0, 0)),
          _bspec((8, H), lambda i: (0, 0)),
          _bspec((8, H), lambda i: (0, 0)),
          _bspec((8, 128), lambda i: (0, 0)),
      ],
      out_specs=_bspec((_EBM, 1), lambda i: (i, 0)),
      out_shape=jax.ShapeDtypeStruct((PEL, 1), jnp.float32),
  )(efl, efr, A1, A2, bl1t, wl2t, bl2t)


def kernel(x, edge_index, edge_label_index, W1, b1, W2, b2, W3, b3,
           Wl1, bl1, Wl2, bl2):
  padn = PE - E
  pad_read = (jnp.arange(padn, dtype=jnp.int32) % 4096)
  pad_dst = N + (jnp.arange(padn, dtype=jnp.int32) % (NP - N))
  src3 = jnp.concatenate([edge_index[0], pad_read]).reshape(NW, NCHUNK, CH)
  dst3 = jnp.concatenate([edge_index[1], pad_dst]).reshape(NW, NCHUNK, CH)
  row3 = jnp.concatenate([edge_label_index[0], pad_read]).reshape(
      NW, NCHUNK, CH)
  col3 = jnp.concatenate([edge_label_index[1], pad_read]).reshape(
      NW, NCHUNK, CH)
  x_pad = jnp.pad(x, ((0, NP - N), (0, 0)))

  b1t = jnp.tile(b1[None, :], (8, 1))
  b2t = jnp.tile(b2[None, :], (8, 1))
  b3t = jnp.tile(b3[None, :], (8, 1))
  bl1t = jnp.tile(bl1[None, :], (8, 1))
  wl2t = jnp.tile(Wl2.T, (8, 1))
  bl2t = jnp.full((8, 128), bl2[0], jnp.float32)
  A1 = Wl1[:OUT]
  A2 = Wl1[OUT:]

  degp = _deg_kernel(dst3)
  degp3 = degp[:, :, None]

  h1 = _tc_b1(x_pad, W1, degp3)
  P1 = _spmm64(h1, src3, dst3)
  h2 = _tc_mid(P1, h1, degp3, b1t, W2, H)
  P2 = _spmm64(h2, src3, dst3)
  h3 = _tc_mid(P2, h2, degp3, b2t, W3, OUT)
  P3 = _spmm32(h3, src3, dst3)
  z_full = _tc_b4(P3, h3, degp3, b3t)

  efl, efr = _decode_gather(z_full, row3, col3)
  link = _tc_mlp(efl, efr, A1, A2, bl1t, wl2t, bl2t)

  return (link[:EL], z_full[:N])


# speedup vs baseline: 11.2107x; 1.0523x over previous
import functools

import jax
import jax.numpy as jnp
from jax import lax
from jax.experimental import pallas as pl
from jax.experimental.pallas import tpu as pltpu
from jax.experimental.pallas import tpu_sc as plsc

N = 10000
NP = 10240
E = 320000
PE = 327680
EL = 320000
PEL = 327680
F_IN = 128
H = 64
OUT = 32

NC = 2
NS = 16
NW = NC * NS
CH = 128
NCHUNK = PE // (NW * CH)
RPT = NP // NS

_MESH = plsc.VectorSubcoreMesh(core_axis_name="c", subcore_axis_name="s",
                               num_cores=NC, num_subcores=NS)
_SC_PARAMS = pltpu.CompilerParams(use_tc_tiling_on_sc=False)


def _fill_zeros(ref, nrows, ncols):
  z16 = jnp.zeros((16,), jnp.float32)
  def body(i, _):
    for j in range(ncols // 16):
      ref[i, pl.ds(j * 16, 16)] = z16
    return 0
  lax.fori_loop(0, nrows, body, 0)


@functools.partial(
    pl.kernel,
    out_type=jax.ShapeDtypeStruct((NC, NP), jnp.float32),
    mesh=_MESH,
    compiler_params=_SC_PARAMS,
    scratch_types=[
        pltpu.VMEM((NCHUNK, CH), jnp.int32),
        pltpu.VMEM((CH,), jnp.float32),
        pltpu.VMEM((RPT,), jnp.float32),
        pltpu.VMEM_SHARED((NP,), jnp.float32),
        pltpu.SemaphoreType.DMA,
    ],
)
def _deg_kernel(dst_hbm, out_hbm, dst_v, ones_v, zb_v, acc, sem):
  c = lax.axis_index("c")
  s = lax.axis_index("s")
  wid = s * NC + c
  pltpu.sync_copy(dst_hbm.at[wid], dst_v)
  one16 = jnp.ones((16,), jnp.float32)
  z16 = jnp.zeros((16,), jnp.float32)
  for j in range(CH // 16):
    ones_v[pl.ds(j * 16, 16)] = one16
  def zbody(i, _):
    zb_v[pl.ds(i * 16, 16)] = z16
    return 0
  lax.fori_loop(0, RPT // 16, zbody, 0)
  pltpu.sync_copy(zb_v, acc.at[pl.ds(s * RPT, RPT)])
  plsc.subcore_barrier()
  def body(j, _):
    pltpu.sync_copy(ones_v, acc.at[dst_v.at[j]], add=True)
    return 0
  lax.fori_loop(0, NCHUNK, body, 0)
  plsc.subcore_barrier()
  pltpu.sync_copy(acc.at[pl.ds(s * RPT, RPT)], zb_v)
  pltpu.sync_copy(zb_v, out_hbm.at[c, pl.ds(s * RPT, RPT)])


def _make_spmm(F):
  @functools.partial(
      pl.kernel,
      out_type=jax.ShapeDtypeStruct((NC, NP, F), jnp.float32),
      mesh=_MESH,
      compiler_params=_SC_PARAMS,
      scratch_types=[
          pltpu.VMEM((NCHUNK, CH), jnp.int32),
          pltpu.VMEM((NCHUNK, CH), jnp.int32),
          pltpu.VMEM((CH, F), jnp.float32),
          pltpu.VMEM((CH, F), jnp.float32),
          pltpu.VMEM((RPT, F), jnp.float32),
          pltpu.VMEM_SHARED((NP, F), jnp.float32),
          pltpu.SemaphoreType.DMA,
          pltpu.SemaphoreType.DMA,
      ],
  )
  def spmm(h_hbm, src_hbm, dst_hbm, out_hbm, src_v, dst_v, rb0, rb1, zb_v,
           acc, sem0, sem1):
    c = lax.axis_index("c")
    s = lax.axis_index("s")
    wid = s * NC + c
    pltpu.sync_copy(src_hbm.at[wid], src_v)
    pltpu.sync_copy(dst_hbm.at[wid], dst_v)
    _fill_zeros(zb_v, CH, F)
    for k in range(RPT // CH):
      pltpu.sync_copy(zb_v.at[pl.ds(0, CH)],
                      acc.at[pl.ds(s * RPT + k * CH, CH)])
    plsc.subcore_barrier()
    pltpu.async_copy(h_hbm.at[src_v.at[0]], rb0, sem0)
    def body(i, _):
      j = 2 * i
      pltpu.async_copy(h_hbm.at[src_v.at[j + 1]], rb1, sem1)
      pltpu.make_async_copy(h_hbm.at[src_v.at[j]], rb0, sem0).wait()
      pltpu.sync_copy(rb0, acc.at[dst_v.at[j]], add=True)
      @pl.when(j + 2 < NCHUNK)
      def _():
        pltpu.async_copy(h_hbm.at[src_v.at[j + 2]], rb0, sem0)
      pltpu.make_async_copy(h_hbm.at[src_v.at[j + 1]], rb1, sem1).wait()
      pltpu.sync_copy(rb1, acc.at[dst_v.at[j + 1]], add=True)
      return 0
    lax.fori_loop(0, NCHUNK // 2, body, 0)
    plsc.subcore_barrier()
    pltpu.sync_copy(acc.at[pl.ds(s * RPT, RPT)], zb_v)
    pltpu.sync_copy(zb_v, out_hbm.at[c, pl.ds(s * RPT, RPT)])
  return spmm


_spmm64 = _make_spmm(H)
_spmm32 = _make_spmm(OUT)


@functools.partial(
    pl.kernel,
    out_type=(jax.ShapeDtypeStruct((PEL, OUT), jnp.float32),
              jax.ShapeDtypeStruct((PEL, OUT), jnp.float32)),
    mesh=_MESH,
    compiler_params=_SC_PARAMS,
    scratch_types=[
        pltpu.VMEM((NCHUNK, CH), jnp.int32),
        pltpu.VMEM((NCHUNK, CH), jnp.int32),
        pltpu.VMEM((CH, OUT), jnp.float32),
        pltpu.VMEM((CH, OUT), jnp.float32),
        pltpu.VMEM((CH, OUT), jnp.float32),
        pltpu.VMEM((CH, OUT), jnp.float32),
        pltpu.SemaphoreType.DMA,
        pltpu.SemaphoreType.DMA,
    ],
)
def _decode_gather(z_hbm, row_hbm, col_hbm, efl_hbm, efr_hbm,
                   row_v, col_v, l0, r0, l1, r1, sem0, sem1):
  c = lax.axis_index("c")
  s = lax.axis_index("s")
  wid = s * NC + c
  base = wid * (NCHUNK * CH)
  pltpu.sync_copy(row_hbm.at[wid], row_v)
  pltpu.sync_copy(col_hbm.at[wid], col_v)
  pltpu.async_copy(z_hbm.at[row_v.at[0]], l0, sem0)
  pltpu.async_copy(z_hbm.at[col_v.at[0]], r0, sem0)
  def body(i, _):
    j = 2 * i
    pltpu.async_copy(z_hbm.at[row_v.at[j + 1]], l1, sem1)
    pltpu.async_copy(z_hbm.at[col_v.at[j + 1]], r1, sem1)
    pltpu.make_async_copy(z_hbm.at[row_v.at[j]], l0, sem0).wait()
    pltpu.make_async_copy(z_hbm.at[col_v.at[j]], r0, sem0).wait()
    pltpu.sync_copy(l0, efl_hbm.at[pl.ds(base + j * CH, CH)])
    pltpu.sync_copy(r0, efr_hbm.at[pl.ds(base + j * CH, CH)])
    @pl.when(j + 2 < NCHUNK)
    def _():
      pltpu.async_copy(z_hbm.at[row_v.at[j + 2]], l0, sem0)
      pltpu.async_copy(z_hbm.at[col_v.at[j + 2]], r0, sem0)
    pltpu.make_async_copy(z_hbm.at[row_v.at[j + 1]], l1, sem1).wait()
    pltpu.make_async_copy(z_hbm.at[col_v.at[j + 1]], r1, sem1).wait()
    pltpu.sync_copy(l1, efl_hbm.at[pl.ds(base + j * CH + CH, CH)])
    pltpu.sync_copy(r1, efr_hbm.at[pl.ds(base + j * CH + CH, CH)])
    return 0
  lax.fori_loop(0, NCHUNK // 2, body, 0)


_BM = 1024
_GRID = NP // _BM


def _dinv_of(degp_ref):
  deg = degp_ref[0] + degp_ref[1] + 1.0
  return lax.rsqrt(deg)


def _b1_body(x_ref, w1_ref, degp_ref, h_ref):
  dinv = _dinv_of(degp_ref)
  h = jnp.dot(x_ref[...], w1_ref[...], preferred_element_type=jnp.float32,
               precision=lax.Precision.HIGHEST)
  h_ref[...] = h * dinv


def _mid_body(p_ref, hprev_ref, degp_ref, b_ref, w_ref, hnext_ref):
  dinv = _dinv_of(degp_ref)
  srt = p_ref[0] + p_ref[1] + hprev_ref[...]
  z = jnp.maximum(srt * dinv + b_ref[0:1, :], 0.0)
  hnext_ref[...] = jnp.dot(z, w_ref[...],
                           preferred_element_type=jnp.float32,
               precision=lax.Precision.HIGHEST) * dinv


def _b4_body(p_ref, hprev_ref, degp_ref, b_ref, z_ref):
  dinv = _dinv_of(degp_ref)
  z_ref[...] = (p_ref[0] + p_ref[1] + hprev_ref[...]) * dinv + b_ref[0:1, :]


def _bspec(shape, im):
  return pl.BlockSpec(shape, im)


def _tc_b1(x_pad, W1, degp3):
  return pl.pallas_call(
      _b1_body,
      grid=(_GRID,),
      in_specs=[
          _bspec((_BM, F_IN), lambda i: (i, 0)),
          _bspec((F_IN, H), lambda i: (0, 0)),
          _bspec((NC, _BM, 1), lambda i: (0, i, 0)),
      ],
      out_specs=_bspec((_BM, H), lambda i: (i, 0)),
      out_shape=jax.ShapeDtypeStruct((NP, H), jnp.float32),
  )(x_pad, W1, degp3)


def _tc_mid(P, hprev, degp3, b2d, W, fout):
  fin = hprev.shape[1]
  return pl.pallas_call(
      _mid_body,
      grid=(_GRID,),
      in_specs=[
          _bspec((NC, _BM, fin), lambda i: (0, i, 0)),
          _bspec((_BM, fin), lambda i: (i, 0)),
          _bspec((NC, _BM, 1), lambda i: (0, i, 0)),
          _bspec((8, fin), lambda i: (0, 0)),
          _bspec((fin, fout), lambda i: (0, 0)),
      ],
      out_specs=_bspec((_BM, fout), lambda i: (i, 0)),
      out_shape=jax.ShapeDtypeStruct((NP, fout), jnp.float32),
  )(P, hprev, degp3, b2d, W)


def _tc_b4(P, hprev, degp3, b2d):
  return pl.pallas_call(
      _b4_body,
      grid=(_GRID,),
      in_specs=[
          _bspec((NC, _BM, OUT), lambda i: (0, i, 0)),
          _bspec((_BM, OUT), lambda i: (i, 0)),
          _bspec((NC, _BM, 1), lambda i: (0, i, 0)),
          _bspec((8, OUT), lambda i: (0, 0)),
      ],
      out_specs=_bspec((_BM, OUT), lambda i: (i, 0)),
      out_shape=jax.ShapeDtypeStruct((NP, OUT), jnp.float32),
  )(P, hprev, degp3, b2d)


_EBM = 2048
_EGRID = PEL // _EBM


def _mlp_body(efl_ref, efr_ref, a1_ref, a2_ref, b1_ref, w2_ref, b2_ref,
              out_ref):
  hl = jnp.dot(efl_ref[...], a1_ref[...], preferred_element_type=jnp.float32,
               precision=lax.Precision.HIGHEST)
  hr = jnp.dot(efr_ref[...], a2_ref[...], preferred_element_type=jnp.float32,
               precision=lax.Precision.HIGHEST)
  hh = jnp.maximum(hl + hr + b1_ref[0:1, :], 0.0)
  w = w2_ref[0:1, :]
  out_ref[...] = jnp.sum(hh * w, axis=1, keepdims=True) + b2_ref[0, 0]


def _tc_mlp(efl, efr, A1, A2, bl1t, wl2t, bl2t):
  return pl.pallas_call(
      _mlp_body,
      grid=(_EGRID,),
      in_specs=[
          _bspec((_EBM, OUT), lambda i: (i, 0)),
          _bspec((_EBM, OUT), lambda i: (i, 0)),
          _bspec((OUT, H), lambda i: (0, 0)),
          _bspec((OUT, H), lambda i: (0, 0)),
          _bspec((8, H), lambda i: (0, 0)),
          _bspec((8, H), lambda i: (0, 0)),
          _bspec((8, 128), lambda i: (0, 0)),
      ],
      out_specs=_bspec((_EBM, 1), lambda i: (i, 0)),
      out_shape=jax.ShapeDtypeStruct((PEL, 1), jnp.float32),
  )(efl, efr, A1, A2, bl1t, wl2t, bl2t)


def kernel(x, edge_index, edge_label_index, W1, b1, W2, b2, W3, b3,
           Wl1, bl1, Wl2, bl2):
  padn = PE - E
  pad_read = (jnp.arange(padn, dtype=jnp.int32) % 4096)
  pad_dst = N + (jnp.arange(padn, dtype=jnp.int32) % (NP - N))
  src3 = jnp.concatenate([edge_index[0], pad_read]).reshape(NW, NCHUNK, CH)
  dst3 = jnp.concatenate([edge_index[1], pad_dst]).reshape(NW, NCHUNK, CH)
  row3 = jnp.concatenate([edge_label_index[0], pad_read]).reshape(
      NW, NCHUNK, CH)
  col3 = jnp.concatenate([edge_label_index[1], pad_read]).reshape(
      NW, NCHUNK, CH)
  x_pad = jnp.pad(x, ((0, NP - N), (0, 0)))

  b1t = jnp.tile(b1[None, :], (8, 1))
  b2t = jnp.tile(b2[None, :], (8, 1))
  b3t = jnp.tile(b3[None, :], (8, 1))
  bl1t = jnp.tile(bl1[None, :], (8, 1))
  wl2t = jnp.tile(Wl2.T, (8, 1))
  bl2t = jnp.full((8, 128), bl2[0], jnp.float32)
  A1 = Wl1[:OUT]
  A2 = Wl1[OUT:]

  degp = _deg_kernel(dst3)
  degp3 = degp[:, :, None]

  h1 = _tc_b1(x_pad, W1, degp3)
  P1 = _spmm64(h1, src3, dst3)
  h2 = _tc_mid(P1, h1, degp3, b1t, W2, H)
  P2 = _spmm64(h2, src3, dst3)
  h3 = _tc_mid(P2, h2, degp3, b2t, W3, OUT)
  P3 = _spmm32(h3, src3, dst3)
  z_full = _tc_b4(P3, h3, degp3, b3t)

  efl, efr = _decode_gather(z_full, row3, col3)
  link = _tc_mlp(efl, efr, A1, A2, bl1t, wl2t, bl2t)

  return (link[:EL], z_full[:N])


# speedup vs baseline: 24.6726x; 2.2008x over previous
import functools

import jax
import jax.numpy as jnp
from jax import lax
from jax.experimental import pallas as pl
from jax.experimental.pallas import tpu as pltpu
from jax.experimental.pallas import tpu_sc as plsc

N = 10000
NP = 10240
E = 320000
PE = 327680
EL = 320000
PEL = 327680
F_IN = 128
H = 64
OUT = 32

NC = 2
NS = 16
NW = NC * NS
CH = 128
NCHUNK = PE // (NW * CH)
RPT = NP // NS

_MESH = plsc.VectorSubcoreMesh(core_axis_name="c", subcore_axis_name="s",
                               num_cores=NC, num_subcores=NS)
_SC_PARAMS = pltpu.CompilerParams(use_tc_tiling_on_sc=False)
_SC_PARAMS_NL = pltpu.CompilerParams(use_tc_tiling_on_sc=False,
                                     needs_layout_passes=False)


def _fill_zeros(ref, nrows, ncols):
  z16 = jnp.zeros((16,), jnp.float32)
  def body(i, _):
    for j in range(ncols // 16):
      ref[i, pl.ds(j * 16, 16)] = z16
    return 0
  lax.fori_loop(0, nrows, body, 0)


@functools.partial(
    pl.kernel,
    out_type=jax.ShapeDtypeStruct((NC, NP), jnp.float32),
    mesh=_MESH,
    compiler_params=_SC_PARAMS,
    scratch_types=[
        pltpu.VMEM((NCHUNK, CH), jnp.int32),
        pltpu.VMEM((CH,), jnp.float32),
        pltpu.VMEM((RPT,), jnp.float32),
        pltpu.VMEM_SHARED((NP,), jnp.float32),
        pltpu.SemaphoreType.DMA,
    ],
)
def _deg_kernel(dst_hbm, out_hbm, dst_v, ones_v, zb_v, acc, sem):
  c = lax.axis_index("c")
  s = lax.axis_index("s")
  wid = s * NC + c
  pltpu.sync_copy(dst_hbm.at[wid], dst_v)
  one16 = jnp.ones((16,), jnp.float32)
  z16 = jnp.zeros((16,), jnp.float32)
  for j in range(CH // 16):
    ones_v[pl.ds(j * 16, 16)] = one16
  def zbody(i, _):
    zb_v[pl.ds(i * 16, 16)] = z16
    return 0
  lax.fori_loop(0, RPT // 16, zbody, 0)
  pltpu.sync_copy(zb_v, acc.at[pl.ds(s * RPT, RPT)])
  plsc.subcore_barrier()
  def body(j, _):
    pltpu.sync_copy(ones_v, acc.at[dst_v.at[j]], add=True)
    return 0
  lax.fori_loop(0, NCHUNK, body, 0)
  plsc.subcore_barrier()
  pltpu.sync_copy(acc.at[pl.ds(s * RPT, RPT)], zb_v)
  pltpu.sync_copy(zb_v, out_hbm.at[c, pl.ds(s * RPT, RPT)])


def _make_spmm(F):
  @functools.partial(
      pl.kernel,
      out_type=jax.ShapeDtypeStruct((NC, NP, F), jnp.float32),
      mesh=_MESH,
      compiler_params=_SC_PARAMS,
      scratch_types=[
          pltpu.VMEM((NCHUNK, CH), jnp.int32),
          pltpu.VMEM((NCHUNK, CH), jnp.int32),
          pltpu.VMEM((CH, F), jnp.float32),
          pltpu.VMEM((CH, F), jnp.float32),
          pltpu.VMEM((RPT, F), jnp.float32),
          pltpu.VMEM_SHARED((NP, F), jnp.float32),
          pltpu.SemaphoreType.DMA,
          pltpu.SemaphoreType.DMA,
      ],
  )
  def spmm(h_hbm, src_hbm, dst_hbm, out_hbm, src_v, dst_v, rb0, rb1, zb_v,
           acc, sem0, sem1):
    c = lax.axis_index("c")
    s = lax.axis_index("s")
    wid = s * NC + c
    pltpu.sync_copy(src_hbm.at[wid], src_v)
    pltpu.sync_copy(dst_hbm.at[wid], dst_v)
    _fill_zeros(zb_v, CH, F)
    for k in range(RPT // CH):
      pltpu.sync_copy(zb_v.at[pl.ds(0, CH)],
                      acc.at[pl.ds(s * RPT + k * CH, CH)])
    plsc.subcore_barrier()
    pltpu.async_copy(h_hbm.at[src_v.at[0]], rb0, sem0)
    def body(i, _):
      j = 2 * i
      pltpu.async_copy(h_hbm.at[src_v.at[j + 1]], rb1, sem1)
      pltpu.make_async_copy(h_hbm.at[src_v.at[j]], rb0, sem0).wait()
      pltpu.sync_copy(rb0, acc.at[dst_v.at[j]], add=True)
      @pl.when(j + 2 < NCHUNK)
      def _():
        pltpu.async_copy(h_hbm.at[src_v.at[j + 2]], rb0, sem0)
      pltpu.make_async_copy(h_hbm.at[src_v.at[j + 1]], rb1, sem1).wait()
      pltpu.sync_copy(rb1, acc.at[dst_v.at[j + 1]], add=True)
      return 0
    lax.fori_loop(0, NCHUNK // 2, body, 0)
    plsc.subcore_barrier()
    pltpu.sync_copy(acc.at[pl.ds(s * RPT, RPT)], zb_v)
    pltpu.sync_copy(zb_v, out_hbm.at[c, pl.ds(s * RPT, RPT)])
  return spmm


_spmm64 = _make_spmm(H)
_spmm32 = _make_spmm(OUT)


def _dot16(bl, br, ws, b2s, mbuf, obuf, g):
  base = g * 16
  for e in range(16):
    m = None
    for k in range(4):
      t = jnp.maximum(bl[base + e, pl.ds(16 * k, 16)]
                      + br[base + e, pl.ds(16 * k, 16)], 0.0) * ws[k]
      m = t if m is None else m + t
    mbuf[e, pl.ds(0, 16)] = m
  iota = lax.iota(jnp.int32, 16)
  r = None
  for cc in range(16):
    colv = plsc.load_gather(mbuf, [iota, jnp.full((16,), cc, jnp.int32)])
    r = colv if r is None else r + colv
  obuf[pl.ds(g * 16, 16)] = r + b2s


@functools.partial(
    pl.kernel,
    out_type=jax.ShapeDtypeStruct((PEL,), jnp.float32),
    mesh=_MESH,
    compiler_params=_SC_PARAMS_NL,
    scratch_types=[
        pltpu.VMEM((NCHUNK, CH), jnp.int32),
        pltpu.VMEM((NCHUNK, CH), jnp.int32),
        pltpu.VMEM((CH, H), jnp.float32),
        pltpu.VMEM((CH, H), jnp.float32),
        pltpu.VMEM((CH, H), jnp.float32),
        pltpu.VMEM((CH, H), jnp.float32),
        pltpu.VMEM((H,), jnp.float32),
        pltpu.VMEM((16,), jnp.float32),
        pltpu.VMEM((16, 16), jnp.float32),
        pltpu.VMEM((CH,), jnp.float32),
        pltpu.SemaphoreType.DMA,
        pltpu.SemaphoreType.DMA,
    ],
)
def _decode(u_hbm, v_hbm, row_hbm, col_hbm, w_hbm, bl2_hbm, out_hbm,
            row_v, col_v, l0, r0, l1, r1, wv, b2v, mbuf, obuf, sem0, sem1):
  c = lax.axis_index("c")
  s = lax.axis_index("s")
  wid = s * NC + c
  base = wid * (NCHUNK * CH)
  pltpu.sync_copy(row_hbm.at[wid], row_v)
  pltpu.sync_copy(col_hbm.at[wid], col_v)
  pltpu.sync_copy(w_hbm, wv)
  pltpu.sync_copy(bl2_hbm, b2v)
  ws = [wv[pl.ds(16 * k, 16)] for k in range(4)]
  b2s = b2v[pl.ds(0, 16)]
  pltpu.async_copy(u_hbm.at[row_v.at[0]], l0, sem0)
  pltpu.async_copy(v_hbm.at[col_v.at[0]], r0, sem0)

  def compute(bl, br, j):
    def gbody(g, _):
      _dot16(bl, br, ws, b2s, mbuf, obuf, g)
      return 0
    lax.fori_loop(0, CH // 16, gbody, 0)
    pltpu.sync_copy(obuf, out_hbm.at[pl.ds(base + j * CH, CH)])

  def body(i, _):
    j = 2 * i
    pltpu.async_copy(u_hbm.at[row_v.at[j + 1]], l1, sem1)
    pltpu.async_copy(v_hbm.at[col_v.at[j + 1]], r1, sem1)
    pltpu.make_async_copy(u_hbm.at[row_v.at[j]], l0, sem0).wait()
    pltpu.make_async_copy(v_hbm.at[col_v.at[j]], r0, sem0).wait()
    compute(l0, r0, j)
    @pl.when(j + 2 < NCHUNK)
    def _():
      pltpu.async_copy(u_hbm.at[row_v.at[j + 2]], l0, sem0)
      pltpu.async_copy(v_hbm.at[col_v.at[j + 2]], r0, sem0)
    pltpu.make_async_copy(u_hbm.at[row_v.at[j + 1]], l1, sem1).wait()
    pltpu.make_async_copy(v_hbm.at[col_v.at[j + 1]], r1, sem1).wait()
    compute(l1, r1, j + 1)
    return 0
  lax.fori_loop(0, NCHUNK // 2, body, 0)


_BM = 1024
_GRID = NP // _BM


def _dinv_of(degp_ref):
  deg = degp_ref[0] + degp_ref[1] + 1.0
  return lax.rsqrt(deg)


def _b1_body(x_ref, w1_ref, degp_ref, h_ref):
  dinv = _dinv_of(degp_ref)
  h = jnp.dot(x_ref[...], w1_ref[...], preferred_element_type=jnp.float32,
               precision=lax.Precision.HIGHEST)
  h_ref[...] = h * dinv


def _mid_body(p_ref, hprev_ref, degp_ref, b_ref, w_ref, hnext_ref):
  dinv = _dinv_of(degp_ref)
  srt = p_ref[0] + p_ref[1] + hprev_ref[...]
  z = jnp.maximum(srt * dinv + b_ref[0:1, :], 0.0)
  hnext_ref[...] = jnp.dot(z, w_ref[...],
                           preferred_element_type=jnp.float32,
               precision=lax.Precision.HIGHEST) * dinv


def _b4_body(p_ref, hprev_ref, degp_ref, b_ref, a1_ref, a2_ref, bl1_ref,
             z_ref, u_ref, v_ref):
  dinv = _dinv_of(degp_ref)
  z = (p_ref[0] + p_ref[1] + hprev_ref[...]) * dinv + b_ref[0:1, :]
  z_ref[...] = z
  u_ref[...] = jnp.dot(z, a1_ref[...], preferred_element_type=jnp.float32,
                       precision=lax.Precision.HIGHEST) + bl1_ref[0:1, :]
  v_ref[...] = jnp.dot(z, a2_ref[...], preferred_element_type=jnp.float32,
                       precision=lax.Precision.HIGHEST)


def _bspec(shape, im):
  return pl.BlockSpec(shape, im)


def _tc_b1(x_pad, W1, degp3):
  return pl.pallas_call(
      _b1_body,
      grid=(_GRID,),
      in_specs=[
          _bspec((_BM, F_IN), lambda i: (i, 0)),
          _bspec((F_IN, H), lambda i: (0, 0)),
          _bspec((NC, _BM, 1), lambda i: (0, i, 0)),
      ],
      out_specs=_bspec((_BM, H), lambda i: (i, 0)),
      out_shape=jax.ShapeDtypeStruct((NP, H), jnp.float32),
  )(x_pad, W1, degp3)


def _tc_mid(P, hprev, degp3, b2d, W, fout):
  fin = hprev.shape[1]
  return pl.pallas_call(
      _mid_body,
      grid=(_GRID,),
      in_specs=[
          _bspec((NC, _BM, fin), lambda i: (0, i, 0)),
          _bspec((_BM, fin), lambda i: (i, 0)),
          _bspec((NC, _BM, 1), lambda i: (0, i, 0)),
          _bspec((8, fin), lambda i: (0, 0)),
          _bspec((fin, fout), lambda i: (0, 0)),
      ],
      out_specs=_bspec((_BM, fout), lambda i: (i, 0)),
      out_shape=jax.ShapeDtypeStruct((NP, fout), jnp.float32),
  )(P, hprev, degp3, b2d, W)


def _tc_b4(P, hprev, degp3, b2d, A1, A2, bl1t):
  return pl.pallas_call(
      _b4_body,
      grid=(_GRID,),
      in_specs=[
          _bspec((NC, _BM, OUT), lambda i: (0, i, 0)),
          _bspec((_BM, OUT), lambda i: (i, 0)),
          _bspec((NC, _BM, 1), lambda i: (0, i, 0)),
          _bspec((8, OUT), lambda i: (0, 0)),
          _bspec((OUT, H), lambda i: (0, 0)),
          _bspec((OUT, H), lambda i: (0, 0)),
          _bspec((8, H), lambda i: (0, 0)),
      ],
      out_specs=[
          _bspec((_BM, OUT), lambda i: (i, 0)),
          _bspec((_BM, H), lambda i: (i, 0)),
          _bspec((_BM, H), lambda i: (i, 0)),
      ],
      out_shape=[
          jax.ShapeDtypeStruct((NP, OUT), jnp.float32),
          jax.ShapeDtypeStruct((NP, H), jnp.float32),
          jax.ShapeDtypeStruct((NP, H), jnp.float32),
      ],
  )(P, hprev, degp3, b2d, A1, A2, bl1t)


def kernel(x, edge_index, edge_label_index, W1, b1, W2, b2, W3, b3,
           Wl1, bl1, Wl2, bl2):
  padn = PE - E
  pad_read = (jnp.arange(padn, dtype=jnp.int32) % 4096)
  pad_dst = N + (jnp.arange(padn, dtype=jnp.int32) % (NP - N))
  src3 = jnp.concatenate([edge_index[0], pad_read]).reshape(NW, NCHUNK, CH)
  dst3 = jnp.concatenate([edge_index[1], pad_dst]).reshape(NW, NCHUNK, CH)
  row3 = jnp.concatenate([edge_label_index[0], pad_read]).reshape(
      NW, NCHUNK, CH)
  col3 = jnp.concatenate([edge_label_index[1], pad_read]).reshape(
      NW, NCHUNK, CH)
  x_pad = jnp.pad(x, ((0, NP - N), (0, 0)))

  b1t = jnp.tile(b1[None, :], (8, 1))
  b2t = jnp.tile(b2[None, :], (8, 1))
  b3t = jnp.tile(b3[None, :], (8, 1))
  bl1t = jnp.tile(bl1[None, :], (8, 1))
  wvec = Wl2[:, 0]
  bl2v = jnp.tile(bl2, (16,))
  A1 = Wl1[:OUT]
  A2 = Wl1[OUT:]

  degp = _deg_kernel(dst3)
  degp3 = degp[:, :, None]

  h1 = _tc_b1(x_pad, W1, degp3)
  P1 = _spmm64(h1, src3, dst3)
  h2 = _tc_mid(P1, h1, degp3, b1t, W2, H)
  P2 = _spmm64(h2, src3, dst3)
  h3 = _tc_mid(P2, h2, degp3, b2t, W3, OUT)
  P3 = _spmm32(h3, src3, dst3)
  z_full, U, V = _tc_b4(P3, h3, degp3, b3t, A1, A2, bl1t)

  link = _decode(U, V, row3, col3, wvec, bl2v)

  return (link[:EL].reshape(EL, 1), z_full[:N])


# speedup vs baseline: 25.9533x; 1.0519x over previous
import functools

import jax
import jax.numpy as jnp
from jax import lax
from jax.experimental import pallas as pl
from jax.experimental.pallas import tpu as pltpu
from jax.experimental.pallas import tpu_sc as plsc

N = 10000
NP = 10240
E = 320000
PE = 327680
EL = 320000
PEL = 327680
F_IN = 128
H = 64
OUT = 32

NC = 2
NS = 16
NW = NC * NS
CH = 128
NCHUNK = PE // (NW * CH)
RPT = NP // NS

_MESH = plsc.VectorSubcoreMesh(core_axis_name="c", subcore_axis_name="s",
                               num_cores=NC, num_subcores=NS)
_SC_PARAMS = pltpu.CompilerParams(use_tc_tiling_on_sc=False)
_SC_PARAMS_NL = pltpu.CompilerParams(use_tc_tiling_on_sc=False,
                                     needs_layout_passes=False)


def _fill_zeros(ref, nrows, ncols):
  z16 = jnp.zeros((16,), jnp.float32)
  def body(i, _):
    for j in range(ncols // 16):
      ref[i, pl.ds(j * 16, 16)] = z16
    return 0
  lax.fori_loop(0, nrows, body, 0)


@functools.partial(
    pl.kernel,
    out_type=jax.ShapeDtypeStruct((NC, NP), jnp.float32),
    mesh=_MESH,
    compiler_params=_SC_PARAMS,
    scratch_types=[
        pltpu.VMEM((NCHUNK, CH), jnp.int32),
        pltpu.VMEM((CH,), jnp.float32),
        pltpu.VMEM((RPT,), jnp.float32),
        pltpu.VMEM_SHARED((NP,), jnp.float32),
        pltpu.SemaphoreType.DMA,
    ],
)
def _deg_kernel(dst_hbm, out_hbm, dst_v, ones_v, zb_v, acc, sem):
  c = lax.axis_index("c")
  s = lax.axis_index("s")
  wid = s * NC + c
  pltpu.sync_copy(dst_hbm.at[wid], dst_v)
  one16 = jnp.ones((16,), jnp.float32)
  z16 = jnp.zeros((16,), jnp.float32)
  for j in range(CH // 16):
    ones_v[pl.ds(j * 16, 16)] = one16
  def zbody(i, _):
    zb_v[pl.ds(i * 16, 16)] = z16
    return 0
  lax.fori_loop(0, RPT // 16, zbody, 0)
  pltpu.sync_copy(zb_v, acc.at[pl.ds(s * RPT, RPT)])
  plsc.subcore_barrier()
  def body(j, _):
    pltpu.sync_copy(ones_v, acc.at[dst_v.at[j]], add=True)
    return 0
  lax.fori_loop(0, NCHUNK, body, 0)
  plsc.subcore_barrier()
  pltpu.sync_copy(acc.at[pl.ds(s * RPT, RPT)], zb_v)
  pltpu.sync_copy(zb_v, out_hbm.at[c, pl.ds(s * RPT, RPT)])


_NB = 5


def _make_spmm(F):
  @functools.partial(
      pl.kernel,
      out_type=jax.ShapeDtypeStruct((NC, NP, F), jnp.float32),
      mesh=_MESH,
      compiler_params=_SC_PARAMS,
      scratch_types=[
          pltpu.VMEM((NCHUNK, CH), jnp.int32),
          pltpu.VMEM((NCHUNK, CH), jnp.int32),
          [pltpu.VMEM((CH, F), jnp.float32) for _ in range(_NB)],
          pltpu.VMEM((CH, F), jnp.float32),
          pltpu.VMEM_SHARED((NP, F), jnp.float32),
          [pltpu.SemaphoreType.DMA for _ in range(_NB)],
          [pltpu.SemaphoreType.DMA for _ in range(_NB)],
      ],
  )
  def spmm(h_hbm, src_hbm, dst_hbm, out_hbm, src_v, dst_v, rb, zb_v,
           acc, gs, ss):
    c = lax.axis_index("c")
    s = lax.axis_index("s")
    wid = s * NC + c
    pltpu.sync_copy(src_hbm.at[wid], src_v)
    pltpu.sync_copy(dst_hbm.at[wid], dst_v)
    _fill_zeros(zb_v, CH, F)
    for k in range(RPT // CH):
      pltpu.sync_copy(zb_v, acc.at[pl.ds(s * RPT + k * CH, CH)])
    plsc.subcore_barrier()
    def body(i, _):
      j = i * _NB
      for k in range(_NB):
        @pl.when(i > 0)
        def _():
          pltpu.make_async_copy(rb[k], acc.at[dst_v.at[j - _NB + k]],
                                ss[k]).wait()
        pltpu.async_copy(h_hbm.at[src_v.at[j + k]], rb[k], gs[k])
      for k in range(_NB):
        pltpu.make_async_copy(h_hbm.at[src_v.at[j + k]], rb[k], gs[k]).wait()
        pltpu.async_copy(rb[k], acc.at[dst_v.at[j + k]], ss[k], add=True)
      return 0
    lax.fori_loop(0, NCHUNK // _NB, body, 0)
    for k in range(_NB):
      pltpu.make_async_copy(rb[k], acc.at[dst_v.at[NCHUNK - _NB + k]],
                            ss[k]).wait()
    plsc.subcore_barrier()
    for k in range(RPT // CH):
      pltpu.sync_copy(acc.at[pl.ds(s * RPT + k * CH, CH)], zb_v)
      pltpu.sync_copy(zb_v, out_hbm.at[c, pl.ds(s * RPT + k * CH, CH)])
  return spmm


_spmm64 = _make_spmm(H)
_spmm32 = _make_spmm(OUT)


def _dot16(bu, bv, ws, b2s, mbuf, obuf, g):
  base = g * 16
  for e in range(16):
    m = None
    for k in range(4):
      t = jnp.maximum(bu[base + e, pl.ds(16 * k, 16)]
                      + bv[base + e, pl.ds(16 * k, 16)], 0.0) * ws[k]
      m = t if m is None else m + t
    mbuf[e, pl.ds(0, 16)] = m
  iota = lax.iota(jnp.int32, 16)
  r = None
  for cc in range(16):
    colv = plsc.load_gather(mbuf, [iota, jnp.full((16,), cc, jnp.int32)])
    r = colv if r is None else r + colv
  obuf[pl.ds(g * 16, 16)] = r + b2s


@functools.partial(
    pl.kernel,
    out_type=jax.ShapeDtypeStruct((PEL,), jnp.float32),
    mesh=_MESH,
    compiler_params=_SC_PARAMS_NL,
    scratch_types=[
        pltpu.VMEM((NCHUNK, CH), jnp.int32),
        pltpu.VMEM((NCHUNK, CH), jnp.int32),
        [pltpu.VMEM((CH, H), jnp.float32) for _ in range(4)],
        [pltpu.VMEM((CH, H), jnp.float32) for _ in range(4)],
        pltpu.VMEM((H,), jnp.float32),
        pltpu.VMEM((16,), jnp.float32),
        pltpu.VMEM((16, 16), jnp.float32),
        pltpu.VMEM((CH,), jnp.float32),
        [pltpu.SemaphoreType.DMA for _ in range(4)],
    ],
)
def _decode(u_hbm, v_hbm, row_hbm, col_hbm, w_hbm, bl2_hbm, out_hbm,
            row_v, col_v, ub, vb, wv, b2v, mbuf, obuf, sems):
  c = lax.axis_index("c")
  s = lax.axis_index("s")
  wid = s * NC + c
  base = wid * (NCHUNK * CH)
  pltpu.sync_copy(row_hbm.at[wid], row_v)
  pltpu.sync_copy(col_hbm.at[wid], col_v)
  pltpu.sync_copy(w_hbm, wv)
  pltpu.sync_copy(bl2_hbm, b2v)
  ws = [wv[pl.ds(16 * k, 16)] for k in range(4)]
  b2s = b2v[pl.ds(0, 16)]

  def compute(bu, bv, j):
    def gbody(g, _):
      _dot16(bu, bv, ws, b2s, mbuf, obuf, g)
      return 0
    lax.fori_loop(0, CH // 16, gbody, 0)
    pltpu.sync_copy(obuf, out_hbm.at[pl.ds(base + j * CH, CH)])

  for k in range(4):
    pltpu.async_copy(u_hbm.at[row_v.at[k]], ub[k], sems[k])
    pltpu.async_copy(v_hbm.at[col_v.at[k]], vb[k], sems[k])
  def body(i, _):
    j = 4 * i
    for k in range(4):
      pltpu.make_async_copy(u_hbm.at[row_v.at[j + k]], ub[k],
                            sems[k]).wait()
      pltpu.make_async_copy(v_hbm.at[col_v.at[j + k]], vb[k],
                            sems[k]).wait()
      compute(ub[k], vb[k], j + k)
      @pl.when(j + k + 4 < NCHUNK)
      def _():
        pltpu.async_copy(u_hbm.at[row_v.at[j + k + 4]], ub[k], sems[k])
        pltpu.async_copy(v_hbm.at[col_v.at[j + k + 4]], vb[k], sems[k])
    return 0
  lax.fori_loop(0, NCHUNK // 4, body, 0)


_BM = 1024
_GRID = NP // _BM


def _dinv_of(degp_ref):
  deg = degp_ref[0] + degp_ref[1] + 1.0
  return lax.rsqrt(deg)


def _b1_body(x_ref, w1_ref, degp_ref, h_ref):
  dinv = _dinv_of(degp_ref)
  h = jnp.dot(x_ref[...], w1_ref[...], preferred_element_type=jnp.float32,
               precision=lax.Precision.HIGHEST)
  h_ref[...] = h * dinv


def _mid_body(p_ref, hprev_ref, degp_ref, b_ref, w_ref, hnext_ref):
  dinv = _dinv_of(degp_ref)
  srt = p_ref[0] + p_ref[1] + hprev_ref[...]
  z = jnp.maximum(srt * dinv + b_ref[0:1, :], 0.0)
  hnext_ref[...] = jnp.dot(z, w_ref[...],
                           preferred_element_type=jnp.float32,
               precision=lax.Precision.HIGHEST) * dinv


def _b4_body(p_ref, hprev_ref, degp_ref, b_ref, a1_ref, a2_ref, bl1_ref,
             z_ref, u_ref, v_ref):
  dinv = _dinv_of(degp_ref)
  z = (p_ref[0] + p_ref[1] + hprev_ref[...]) * dinv + b_ref[0:1, :]
  z_ref[...] = z
  u_ref[...] = jnp.dot(z, a1_ref[...], preferred_element_type=jnp.float32,
                       precision=lax.Precision.HIGHEST) + bl1_ref[0:1, :]
  v_ref[...] = jnp.dot(z, a2_ref[...], preferred_element_type=jnp.float32,
                       precision=lax.Precision.HIGHEST)


def _bspec(shape, im):
  return pl.BlockSpec(shape, im)


def _tc_b1(x_pad, W1, degp3):
  return pl.pallas_call(
      _b1_body,
      grid=(_GRID,),
      in_specs=[
          _bspec((_BM, F_IN), lambda i: (i, 0)),
          _bspec((F_IN, H), lambda i: (0, 0)),
          _bspec((NC, _BM, 1), lambda i: (0, i, 0)),
      ],
      out_specs=_bspec((_BM, H), lambda i: (i, 0)),
      out_shape=jax.ShapeDtypeStruct((NP, H), jnp.float32),
  )(x_pad, W1, degp3)


def _tc_mid(P, hprev, degp3, b2d, W, fout):
  fin = hprev.shape[1]
  return pl.pallas_call(
      _mid_body,
      grid=(_GRID,),
      in_specs=[
          _bspec((NC, _BM, fin), lambda i: (0, i, 0)),
          _bspec((_BM, fin), lambda i: (i, 0)),
          _bspec((NC, _BM, 1), lambda i: (0, i, 0)),
          _bspec((8, fin), lambda i: (0, 0)),
          _bspec((fin, fout), lambda i: (0, 0)),
      ],
      out_specs=_bspec((_BM, fout), lambda i: (i, 0)),
      out_shape=jax.ShapeDtypeStruct((NP, fout), jnp.float32),
  )(P, hprev, degp3, b2d, W)


def _tc_b4(P, hprev, degp3, b2d, A1, A2, bl1t):
  return pl.pallas_call(
      _b4_body,
      grid=(_GRID,),
      in_specs=[
          _bspec((NC, _BM, OUT), lambda i: (0, i, 0)),
          _bspec((_BM, OUT), lambda i: (i, 0)),
          _bspec((NC, _BM, 1), lambda i: (0, i, 0)),
          _bspec((8, OUT), lambda i: (0, 0)),
          _bspec((OUT, H), lambda i: (0, 0)),
          _bspec((OUT, H), lambda i: (0, 0)),
          _bspec((8, H), lambda i: (0, 0)),
      ],
      out_specs=[
          _bspec((_BM, OUT), lambda i: (i, 0)),
          _bspec((_BM, H), lambda i: (i, 0)),
          _bspec((_BM, H), lambda i: (i, 0)),
      ],
      out_shape=[
          jax.ShapeDtypeStruct((NP, OUT), jnp.float32),
          jax.ShapeDtypeStruct((NP, H), jnp.float32),
          jax.ShapeDtypeStruct((NP, H), jnp.float32),
      ],
  )(P, hprev, degp3, b2d, A1, A2, bl1t)


def kernel(x, edge_index, edge_label_index, W1, b1, W2, b2, W3, b3,
           Wl1, bl1, Wl2, bl2):
  padn = PE - E
  pad_read = (jnp.arange(padn, dtype=jnp.int32) % 4096)
  pad_dst = N + (jnp.arange(padn, dtype=jnp.int32) % (NP - N))
  src3 = jnp.concatenate([edge_index[0], pad_read]).reshape(NW, NCHUNK, CH)
  dst3 = jnp.concatenate([edge_index[1], pad_dst]).reshape(NW, NCHUNK, CH)
  row3 = jnp.concatenate([edge_label_index[0], pad_read]).reshape(
      NW, NCHUNK, CH)
  col3 = jnp.concatenate([edge_label_index[1], pad_read]).reshape(
      NW, NCHUNK, CH)
  x_pad = jnp.pad(x, ((0, NP - N), (0, 0)))

  b1t = jnp.tile(b1[None, :], (8, 1))
  b2t = jnp.tile(b2[None, :], (8, 1))
  b3t = jnp.tile(b3[None, :], (8, 1))
  bl1t = jnp.tile(bl1[None, :], (8, 1))
  wvec = Wl2[:, 0]
  bl2v = jnp.tile(bl2, (16,))
  A1 = Wl1[:OUT]
  A2 = Wl1[OUT:]

  degp = _deg_kernel(dst3)
  degp3 = degp[:, :, None]

  h1 = _tc_b1(x_pad, W1, degp3)
  P1 = _spmm64(h1, src3, dst3)
  h2 = _tc_mid(P1, h1, degp3, b1t, W2, H)
  P2 = _spmm64(h2, src3, dst3)
  h3 = _tc_mid(P2, h2, degp3, b2t, W3, OUT)
  P3 = _spmm32(h3, src3, dst3)
  z_full, U, V = _tc_b4(P3, h3, degp3, b3t, A1, A2, bl1t)

  link = _decode(U, V, row3, col3, wvec, bl2v)

  return (link[:EL].reshape(EL, 1), z_full[:N])


# speedup vs baseline: 26.2328x; 1.0108x over previous
import functools

import jax
import jax.numpy as jnp
from jax import lax
from jax.experimental import pallas as pl
from jax.experimental.pallas import tpu as pltpu
from jax.experimental.pallas import tpu_sc as plsc

N = 10000
NP = 10240
E = 320000
PE = 327680
EL = 320000
PEL = 327680
F_IN = 128
H = 64
OUT = 32

NC = 2
NS = 16
NW = NC * NS
CH = 128
NCHUNK = PE // (NW * CH)
RPT = NP // NS

_MESH = plsc.VectorSubcoreMesh(core_axis_name="c", subcore_axis_name="s",
                               num_cores=NC, num_subcores=NS)
_SC_PARAMS = pltpu.CompilerParams(use_tc_tiling_on_sc=False)
_SC_PARAMS_NL = pltpu.CompilerParams(use_tc_tiling_on_sc=False,
                                     needs_layout_passes=False)


def _fill_zeros(ref, nrows, ncols):
  z16 = jnp.zeros((16,), jnp.float32)
  def body(i, _):
    for j in range(ncols // 16):
      ref[i, pl.ds(j * 16, 16)] = z16
    return 0
  lax.fori_loop(0, nrows, body, 0)


@functools.partial(
    pl.kernel,
    out_type=jax.ShapeDtypeStruct((NC, NP), jnp.float32),
    mesh=_MESH,
    compiler_params=_SC_PARAMS,
    scratch_types=[
        pltpu.VMEM((NCHUNK, CH), jnp.int32),
        pltpu.VMEM((CH,), jnp.float32),
        pltpu.VMEM((RPT,), jnp.float32),
        pltpu.VMEM_SHARED((NP,), jnp.float32),
        pltpu.SemaphoreType.DMA,
    ],
)
def _deg_kernel(dst_hbm, out_hbm, dst_v, ones_v, zb_v, acc, sem):
  c = lax.axis_index("c")
  s = lax.axis_index("s")
  wid = s * NC + c
  pltpu.sync_copy(dst_hbm.at[wid], dst_v)
  one16 = jnp.ones((16,), jnp.float32)
  z16 = jnp.zeros((16,), jnp.float32)
  for j in range(CH // 16):
    ones_v[pl.ds(j * 16, 16)] = one16
  def zbody(i, _):
    zb_v[pl.ds(i * 16, 16)] = z16
    return 0
  lax.fori_loop(0, RPT // 16, zbody, 0)
  pltpu.sync_copy(zb_v, acc.at[pl.ds(s * RPT, RPT)])
  plsc.subcore_barrier()
  def body(j, _):
    pltpu.sync_copy(ones_v, acc.at[dst_v.at[j]], add=True)
    return 0
  lax.fori_loop(0, NCHUNK, body, 0)
  plsc.subcore_barrier()
  pltpu.sync_copy(acc.at[pl.ds(s * RPT, RPT)], zb_v)
  pltpu.sync_copy(zb_v, out_hbm.at[c, pl.ds(s * RPT, RPT)])


_NB = 5


def _make_spmm(F):
  @functools.partial(
      pl.kernel,
      out_type=jax.ShapeDtypeStruct((NC, NP, F), jnp.float32),
      mesh=_MESH,
      compiler_params=_SC_PARAMS,
      scratch_types=[
          pltpu.VMEM((NCHUNK, CH), jnp.int32),
          pltpu.VMEM((NCHUNK, CH), jnp.int32),
          [pltpu.VMEM((CH, F), jnp.float32) for _ in range(_NB)],
          pltpu.VMEM((CH, F), jnp.float32),
          pltpu.VMEM_SHARED((NP, F), jnp.float32),
          [pltpu.SemaphoreType.DMA for _ in range(_NB)],
          [pltpu.SemaphoreType.DMA for _ in range(_NB)],
      ],
  )
  def spmm(h_hbm, src_hbm, dst_hbm, out_hbm, src_v, dst_v, rb, zb_v,
           acc, gs, ss):
    c = lax.axis_index("c")
    s = lax.axis_index("s")
    wid = s * NC + c
    pltpu.sync_copy(src_hbm.at[wid], src_v)
    pltpu.sync_copy(dst_hbm.at[wid], dst_v)
    _fill_zeros(zb_v, CH, F)
    for k in range(RPT // CH):
      pltpu.sync_copy(zb_v, acc.at[pl.ds(s * RPT + k * CH, CH)])
    plsc.subcore_barrier()
    def body(i, _):
      j = i * _NB
      for k in range(_NB):
        @pl.when(i > 0)
        def _():
          pltpu.make_async_copy(rb[k], acc.at[dst_v.at[j - _NB + k]],
                                ss[k]).wait()
        pltpu.async_copy(h_hbm.at[src_v.at[j + k]], rb[k], gs[k])
      for k in range(_NB):
        pltpu.make_async_copy(h_hbm.at[src_v.at[j + k]], rb[k], gs[k]).wait()
        pltpu.async_copy(rb[k], acc.at[dst_v.at[j + k]], ss[k], add=True)
      return 0
    lax.fori_loop(0, NCHUNK // _NB, body, 0)
    for k in range(_NB):
      pltpu.make_async_copy(rb[k], acc.at[dst_v.at[NCHUNK - _NB + k]],
                            ss[k]).wait()
    plsc.subcore_barrier()
    for k in range(RPT // CH):
      pltpu.sync_copy(acc.at[pl.ds(s * RPT + k * CH, CH)], zb_v)
      pltpu.sync_copy(zb_v, out_hbm.at[c, pl.ds(s * RPT + k * CH, CH)])
  return spmm


_spmm64 = _make_spmm(H)
_spmm32 = _make_spmm(OUT)


def _dot16(bu, bv, ws, b2s, mbuf, obuf, g, off):
  base = g * 16
  for e in range(16):
    m = None
    for k in range(4):
      t = jnp.maximum(bu[base + e, pl.ds(16 * k, 16)]
                      + bv[base + e, pl.ds(16 * k, 16)], 0.0) * ws[k]
      m = t if m is None else m + t
    mbuf[e, pl.ds(0, 16)] = m
  iota = lax.iota(jnp.int32, 16)
  r = None
  for cc in range(16):
    colv = plsc.load_gather(mbuf, [iota, jnp.full((16,), cc, jnp.int32)])
    r = colv if r is None else r + colv
  obuf[pl.ds(off, 16)] = r + b2s


@functools.partial(
    pl.kernel,
    out_type=jax.ShapeDtypeStruct((PEL,), jnp.float32),
    mesh=_MESH,
    compiler_params=_SC_PARAMS_NL,
    scratch_types=[
        pltpu.VMEM((NCHUNK, CH), jnp.int32),
        pltpu.VMEM((NCHUNK, CH), jnp.int32),
        [pltpu.VMEM((CH, H), jnp.float32) for _ in range(4)],
        [pltpu.VMEM((CH, H), jnp.float32) for _ in range(4)],
        pltpu.VMEM((H,), jnp.float32),
        pltpu.VMEM((16,), jnp.float32),
        pltpu.VMEM((16, 16), jnp.float32),
        pltpu.VMEM((NCHUNK * CH,), jnp.float32),
        [pltpu.SemaphoreType.DMA for _ in range(4)],
    ],
)
def _decode(u_hbm, v_hbm, row_hbm, col_hbm, w_hbm, bl2_hbm, out_hbm,
            row_v, col_v, ub, vb, wv, b2v, mbuf, obuf, sems):
  c = lax.axis_index("c")
  s = lax.axis_index("s")
  wid = s * NC + c
  base = wid * (NCHUNK * CH)
  pltpu.sync_copy(row_hbm.at[wid], row_v)
  pltpu.sync_copy(col_hbm.at[wid], col_v)
  pltpu.sync_copy(w_hbm, wv)
  pltpu.sync_copy(bl2_hbm, b2v)
  ws = [wv[pl.ds(16 * k, 16)] for k in range(4)]
  b2s = b2v[pl.ds(0, 16)]

  def compute(bu, bv, j):
    def gbody(g, _):
      _dot16(bu, bv, ws, b2s, mbuf, obuf, g, j * CH + g * 16)
      return 0
    lax.fori_loop(0, CH // 16, gbody, 0)

  for k in range(4):
    pltpu.async_copy(u_hbm.at[row_v.at[k]], ub[k], sems[k])
    pltpu.async_copy(v_hbm.at[col_v.at[k]], vb[k], sems[k])
  def body(i, _):
    j = 4 * i
    for k in range(4):
      pltpu.make_async_copy(u_hbm.at[row_v.at[j + k]], ub[k],
                            sems[k]).wait()
      pltpu.make_async_copy(v_hbm.at[col_v.at[j + k]], vb[k],
                            sems[k]).wait()
      compute(ub[k], vb[k], j + k)
      @pl.when(j + k + 4 < NCHUNK)
      def _():
        pltpu.async_copy(u_hbm.at[row_v.at[j + k + 4]], ub[k], sems[k])
        pltpu.async_copy(v_hbm.at[col_v.at[j + k + 4]], vb[k], sems[k])
    return 0
  lax.fori_loop(0, NCHUNK // 4, body, 0)


_BM = 1024
_GRID = NP // _BM


def _dinv_of(degp_ref):
  deg = degp_ref[0] + degp_ref[1] + 1.0
  return lax.rsqrt(deg)


def _b1_body(x_ref, w1_ref, degp_ref, h_ref):
  dinv = _dinv_of(degp_ref)
  h = jnp.dot(x_ref[...], w1_ref[...], preferred_element_type=jnp.float32,
               precision=lax.Precision.HIGHEST)
  h_ref[...] = h * dinv


def _mid_body(p_ref, hprev_ref, degp_ref, b_ref, w_ref, hnext_ref):
  dinv = _dinv_of(degp_ref)
  srt = p_ref[0] + p_ref[1] + hprev_ref[...]
  z = jnp.maximum(srt * dinv + b_ref[0:1, :], 0.0)
  hnext_ref[...] = jnp.dot(z, w_ref[...],
                           preferred_element_type=jnp.float32,
               precision=lax.Precision.HIGHEST) * dinv


def _b4_body(p_ref, hprev_ref, degp_ref, b_ref, a1_ref, a2_ref, bl1_ref,
             z_ref, u_ref, v_ref):
  dinv = _dinv_of(degp_ref)
  z = (p_ref[0] + p_ref[1] + hprev_ref[...]) * dinv + b_ref[0:1, :]
  z_ref[...] = z
  u_ref[...] = jnp.dot(z, a1_ref[...], preferred_element_type=jnp.float32,
                       precision=lax.Precision.HIGHEST) + bl1_ref[0:1, :]
  v_ref[...] = jnp.dot(z, a2_ref[...], preferred_element_type=jnp.float32,
                       precision=lax.Precision.HIGHEST)


def _bspec(shape, im):
  return pl.BlockSpec(shape, im)


def _tc_b1(x_pad, W1, degp3):
  return pl.pallas_call(
      _b1_body,
      grid=(_GRID,),
      in_specs=[
          _bspec((_BM, F_IN), lambda i: (i, 0)),
          _bspec((F_IN, H), lambda i: (0, 0)),
          _bspec((NC, _BM, 1), lambda i: (0, i, 0)),
      ],
      out_specs=_bspec((_BM, H), lambda i: (i, 0)),
      out_shape=jax.ShapeDtypeStruct((NP, H), jnp.float32),
  )(x_pad, W1, degp3)


def _tc_mid(P, hprev, degp3, b2d, W, fout):
  fin = hprev.shape[1]
  return pl.pallas_call(
      _mid_body,
      grid=(_GRID,),
      in_specs=[
          _bspec((NC, _BM, fin), lambda i: (0, i, 0)),
          _bspec((_BM, fin), lambda i: (i, 0)),
          _bspec((NC, _BM, 1), lambda i: (0, i, 0)),
          _bspec((8, fin), lambda i: (0, 0)),
          _bspec((fin, fout), lambda i: (0, 0)),
      ],
      out_specs=_bspec((_BM, fout), lambda i: (i, 0)),
      out_shape=jax.ShapeDtypeStruct((NP, fout), jnp.float32),
  )(P, hprev, degp3, b2d, W)


def _tc_b4(P, hprev, degp3, b2d, A1, A2, bl1t):
  return pl.pallas_call(
      _b4_body,
      grid=(_GRID,),
      in_specs=[
          _bspec((NC, _BM, OUT), lambda i: (0, i, 0)),
          _bspec((_BM, OUT), lambda i: (i, 0)),
          _bspec((NC, _BM, 1), lambda i: (0, i, 0)),
          _bspec((8, OUT), lambda i: (0, 0)),
          _bspec((OUT, H), lambda i: (0, 0)),
          _bspec((OUT, H), lambda i: (0, 0)),
          _bspec((8, H), lambda i: (0, 0)),
      ],
      out_specs=[
          _bspec((_BM, OUT), lambda i: (i, 0)),
          _bspec((_BM, H), lambda i: (i, 0)),
          _bspec((_BM, H), lambda i: (i, 0)),
      ],
      out_shape=[
          jax.ShapeDtypeStruct((NP, OUT), jnp.float32),
          jax.ShapeDtypeStruct((NP, H), jnp.float32),
          jax.ShapeDtypeStruct((NP, H), jnp.float32),
      ],
  )(P, hprev, degp3, b2d, A1, A2, bl1t)


def kernel(x, edge_index, edge_label_index, W1, b1, W2, b2, W3, b3,
           Wl1, bl1, Wl2, bl2):
  padn = PE - E
  pad_read = (jnp.arange(padn, dtype=jnp.int32) % 4096)
  pad_dst = N + (jnp.arange(padn, dtype=jnp.int32) % (NP - N))
  src3 = jnp.concatenate([edge_index[0], pad_read]).reshape(NW, NCHUNK, CH)
  dst3 = jnp.concatenate([edge_index[1], pad_dst]).reshape(NW, NCHUNK, CH)
  row3 = jnp.concatenate([edge_label_index[0], pad_read]).reshape(
      NW, NCHUNK, CH)
  col3 = jnp.concatenate([edge_label_index[1], pad_read]).reshape(
      NW, NCHUNK, CH)
  x_pad = jnp.pad(x, ((0, NP - N), (0, 0)))

  b1t = jnp.tile(b1[None, :], (8, 1))
  b2t = jnp.tile(b2[None, :], (8, 1))
  b3t = jnp.tile(b3[None, :], (8, 1))
  bl1t = jnp.tile(bl1[None, :], (8, 1))
  wvec = Wl2[:, 0]
  bl2v = jnp.tile(bl2, (16,))
  A1 = Wl1[:OUT]
  A2 = Wl1[OUT:]

  degp = _deg_kernel(dst3)
  degp3 = degp[:, :, None]

  h1 = _tc_b1(x_pad, W1, degp3)
  P1 = _spmm64(h1, src3, dst3)
  h2 = _tc_mid(P1, h1, degp3, b1t, W2, H)
  P2 = _spmm64(h2, src3, dst3)
  h3 = _tc_mid(P2, h2, degp3, b2t, W3, OUT)
  P3 = _spmm32(h3, src3, dst3)
  z_full, U, V = _tc_b4(P3, h3, degp3, b3t, A1, A2, bl1t)

  link = _decode(U, V, row3, col3, wvec, bl2v)

  return (link[:EL].reshape(EL, 1), z_full[:N])
